# baseline (speedup 1.0000x reference)
import functools

import jax
import jax.numpy as jnp
from jax import lax
from jax.experimental import pallas as pl
from jax.experimental.pallas import tpu as pltpu
from jax.experimental.pallas import tpu_sc as plsc

NC = 2
NS = 16
NW = NC * NS

KA = 16
KB = 8


@functools.lru_cache(maxsize=None)
def _build(B, V, D):
    bpw = B // NW
    cyc = KA + KB
    n_full = bpw // cyc
    rem = bpw - n_full * cyc
    assert B % NW == 0 and bpw % 8 == 0 and rem in (0, KB)

    mesh = plsc.VectorSubcoreMesh(core_axis_name="c", subcore_axis_name="s")

    @functools.partial(
        pl.kernel,
        mesh=mesh,
        out_type=jax.ShapeDtypeStruct((B, D), jnp.float32),
        scratch_types=[
            pltpu.VMEM((bpw,), jnp.int32),
            pltpu.VMEM((KA, D), jnp.float32),
            pltpu.VMEM((KB, D), jnp.float32),
            pltpu.SemaphoreType.DMA,
            pltpu.SemaphoreType.DMA,
            pltpu.SemaphoreType.DMA,
            pltpu.SemaphoreType.DMA,
        ],
    )
    def emb(idx_hbm, tab_hbm, out_hbm, idx_v, buf_a, buf_b, ga, gb, wa, wb):
        wid = lax.axis_index("s") * NC + lax.axis_index("c")
        base = wid * bpw
        pltpu.sync_copy(idx_hbm.at[pl.ds(base, bpw)], idx_v)

        def start_gather(buf, sem, off, k):
            off = pl.multiple_of(off, 8)
            pltpu.async_copy(
                tab_hbm.at[idx_v.at[pl.ds(off, k)]], buf.at[pl.ds(0, k)], sem)

        def wait_gather(buf, sem, k):
            pltpu.make_async_copy(
                tab_hbm.at[idx_v.at[pl.ds(0, k)]], buf.at[pl.ds(0, k)],
                sem).wait()

        def start_write(buf, sem, off, k):
            off = pl.multiple_of(base + off, 8)
            pltpu.async_copy(
                buf.at[pl.ds(0, k)], out_hbm.at[pl.ds(off, k)], sem)

        def wait_write(buf, sem, k):
            pltpu.make_async_copy(
                buf.at[pl.ds(0, k)], out_hbm.at[pl.ds(0, k)], sem).wait()

        start_gather(buf_a, ga, 0, KA)
        start_gather(buf_b, gb, KA, KB)

        @pl.loop(0, n_full - 1)
        def _(i):
            off = i * cyc
            wait_gather(buf_a, ga, KA)
            start_write(buf_a, wa, off, KA)
            wait_write(buf_a, wa, KA)
            start_gather(buf_a, ga, off + cyc, KA)
            wait_gather(buf_b, gb, KB)
            start_write(buf_b, wb, off + KA, KB)
            wait_write(buf_b, wb, KB)
            start_gather(buf_b, gb, off + cyc + KA, KB)

        last = (n_full - 1) * cyc
        wait_gather(buf_a, ga, KA)
        start_write(buf_a, wa, last, KA)
        if rem:
            wait_write(buf_a, wa, KA)
            start_gather(buf_a, ga, last + cyc, rem)
        wait_gather(buf_b, gb, KB)
        start_write(buf_b, wb, last + KA, KB)
        if rem:
            wait_gather(buf_a, ga, rem)
            start_write(buf_a, wa, last + cyc, rem)
            wait_write(buf_a, wa, rem)
        else:
            wait_write(buf_a, wa, KA)
        wait_write(buf_b, wb, KB)

    return emb


def kernel(input_ids, embed_weight):
    V, D = embed_weight.shape
    idx = input_ids.reshape(-1).astype(jnp.int32)
    B = idx.shape[0]
    out = _build(B, V, D)(idx, embed_weight)
    return out.reshape(input_ids.shape + (D,))


# speedup vs baseline: 1.0243x; 1.0243x over previous
import functools
import jax
import jax.numpy as jnp
from jax import lax
from jax.experimental import pallas as pl
from jax.experimental.pallas import tpu as pltpu
from jax.experimental.pallas import tpu_sc as plsc

NC, NS = 2, 16
NW = NC * NS
K = 8
H = 4

@functools.lru_cache(maxsize=None)
def _build(B, V, D):
    bpw = B // NW
    chunks = bpw // K
    mesh = plsc.VectorSubcoreMesh(core_axis_name="c", subcore_axis_name="s")

    @functools.partial(
        pl.kernel, mesh=mesh,
        out_type=jax.ShapeDtypeStruct((B, D), jnp.float32),
        scratch_types=(
            [pltpu.VMEM((bpw,), jnp.int32),
             pltpu.VMEM((2, K, D), jnp.float32),
             pltpu.MemorySpace.VMEM_SHARED((NS, 2, H, D), jnp.float32)]
            + [pltpu.SemaphoreType.DMA] * 5
        ),
    )
    def emb(idx_hbm, tab_hbm, out_hbm, idx_v, bufs, shr, *sems):
        gsems = sems[0:2]
        xsem = sems[2]
        wsems = sems[3:5]
        wid = lax.axis_index("s") * NC + lax.axis_index("c")
        sid = lax.axis_index("s")
        base = wid * bpw
        pltpu.sync_copy(idx_hbm.at[pl.ds(base, bpw)], idx_v)

        def start_gather(b, g):
            off = pl.multiple_of(g * K, 8)
            pltpu.async_copy(
                tab_hbm.at[idx_v.at[pl.ds(off, K)]], bufs.at[b], gsems[b])

        def wait_gather(b):
            pltpu.make_async_copy(
                tab_hbm.at[idx_v.at[pl.ds(0, K)]], bufs.at[b], gsems[b]).wait()

        def do_x(b, h):
            src = bufs.at[b].at[pl.ds(h * H, H)]
            pltpu.async_copy(src, shr.at[sid, h], xsem)
            pltpu.make_async_copy(src, shr.at[sid, h], xsem).wait()

        def start_write(h, g):
            off = pl.multiple_of(base + g * K + h * H, 4)
            pltpu.async_copy(shr.at[sid, h], out_hbm.at[pl.ds(off, H)],
                             wsems[h])

        def wait_write(h):
            pltpu.make_async_copy(shr.at[sid, h],
                                  out_hbm.at[pl.ds(0, H)], wsems[h]).wait()

        def process(b, g, first=False, nxt=True):
            wait_gather(b)
            for h in range(2):
                if not first:
                    wait_write(h)
                do_x(b, h)
                start_write(h, g)
            if nxt:
                start_gather(b, g + 2)

        start_gather(0, 0)
        start_gather(1, 1)
        process(0, 0, first=True)

        @pl.loop(0, (chunks - 4) // 2)
        def _(i):
            process(1, 2 * i + 1)
            process(0, 2 * i + 2)

        process(1, chunks - 3)
        process(0, chunks - 2, nxt=False)
        process(1, chunks - 1, nxt=False)
        for h in range(2):
            wait_write(h)

    return emb


def kernel(input_ids, embed_weight):
    V, D = embed_weight.shape
    idx = input_ids.reshape(-1).astype(jnp.int32)
    B = idx.shape[0]
    out = _build(B, V, D)(idx, embed_weight)
    return out.reshape(input_ids.shape + (D,))
